# baseline (speedup 1.0000x reference)
import functools

import jax
import jax.numpy as jnp
from jax import lax
from jax.experimental import pallas as pl
from jax.experimental.pallas import tpu as pltpu
from jax.experimental.pallas import tpu_sc as plsc

VOCAB = 1000000
SEQ_LEN = 200
EMBED_DIM = 64
BATCH = 4096

NC, NS, L = 2, 16, 16
NW = NC * NS
TOTAL = BATCH * SEQ_LEN
ROWS_PER_W = TOTAL // NW
G = 100
CHUNK = 800
N_G = CHUNK // G
N_CHUNKS = ROWS_PER_W // CHUNK
IDX_ROWS_PER_CHUNK = CHUNK // G

_SCALE = 8.0


def _emb_kernel(idx_hbm, tok_hbm, pos_hbm, out_hbm, pos_v, idx_v, rows_v, sem):
    wid = lax.axis_index("s") * NC + lax.axis_index("c")
    base = wid * ROWS_PER_W

    pltpu.sync_copy(pos_hbm, pos_v)

    def chunk_body(ci, _):
        row0 = base + ci * CHUNK
        idx_row0 = pl.multiple_of(row0 // G, 8)
        pltpu.sync_copy(idx_hbm.at[pl.ds(idx_row0, N_G)], idx_v)
        cps = [
            pltpu.async_copy(
                tok_hbm.at[idx_v.at[g]], rows_v.at[pl.ds(g * G, G)], sem
            )
            for g in range(N_G)
        ]
        for cp in cps:
            cp.wait()

        def row_body(r, carry):
            p = lax.rem(r, SEQ_LEN)
            for c in range(EMBED_DIM // L):
                sl = pl.ds(c * L, L)
                rows_v[r, sl] = rows_v[r, sl] * _SCALE + pos_v[p, sl]
            return carry
        lax.fori_loop(0, CHUNK, row_body, 0)

        pltpu.sync_copy(rows_v, out_hbm.at[pl.ds(row0, CHUNK)])
        return _

    lax.fori_loop(0, N_CHUNKS, chunk_body, 0)


@jax.jit
def _run(idx2d, token_table, position_table):
    mesh = plsc.VectorSubcoreMesh(
        core_axis_name="c", subcore_axis_name="s", num_cores=NC, num_subcores=NS
    )
    kern = functools.partial(
        pl.kernel,
        out_type=jax.ShapeDtypeStruct((TOTAL, EMBED_DIM), jnp.float32),
        mesh=mesh,
        scratch_types=[
            pltpu.VMEM((SEQ_LEN, EMBED_DIM), jnp.float32),
            pltpu.VMEM((N_G, G), jnp.int32),
            pltpu.VMEM((CHUNK, EMBED_DIM), jnp.float32),
            pltpu.SemaphoreType.DMA,
        ],
        compiler_params=pltpu.CompilerParams(use_tc_tiling_on_sc=False),
    )(_emb_kernel)
    return kern(idx2d, token_table, position_table)


def kernel(inputs, token_table, position_table):
    idx2d = inputs.reshape(TOTAL // G, G)
    out = _run(idx2d, token_table, position_table)
    return out.reshape(BATCH, SEQ_LEN, EMBED_DIM)


# speedup vs baseline: 1.3795x; 1.3795x over previous
import functools

import jax
import jax.numpy as jnp
from jax import lax
from jax.experimental import pallas as pl
from jax.experimental.pallas import tpu as pltpu
from jax.experimental.pallas import tpu_sc as plsc

VOCAB = 1000000
SEQ_LEN = 200
EMBED_DIM = 64
BATCH = 4096

NC, NS, L = 2, 16, 16
NW = NC * NS
B_PER_W = BATCH // NW
CB = 4
PAIR = 2 * CB
N_PAIRS = B_PER_W // PAIR
G = 40
GPR = SEQ_LEN // G

_SCALE = 8.0


def _emb_kernel(idx_hbm, tok_hbm, pos_hbm, out_hbm,
                pos_v, idx_v, rows_a, rows_b, sg_a, sg_b, ss_a, ss_b):
    wid = lax.axis_index("s") * NC + lax.axis_index("c")
    base = wid * B_PER_W

    pltpu.sync_copy(pos_hbm, pos_v)

    bufs = (rows_a, rows_b)
    gsems = (sg_a, sg_b)
    ssems = (ss_a, ss_b)

    def gather_chunk(half, pb):
        buf, sem = bufs[half], gsems[half]
        for h in range(GPR * CB):
            pltpu.async_copy(
                tok_hbm.at[idx_v.at[half * CB + h // GPR, pl.ds((h % GPR) * G, G)]],
                buf.at[h // GPR, pl.ds((h % GPR) * G, G), :],
                sem,
            )

    def drain_gather(half):
        buf, sem = bufs[half], gsems[half]
        for h in range(GPR * CB):
            pltpu.make_async_copy(
                tok_hbm.at[idx_v.at[half * CB + h // GPR, pl.ds((h % GPR) * G, G)]],
                buf.at[h // GPR, pl.ds((h % GPR) * G, G), :],
                sem,
            ).wait()

    def drain_scatter(half, dst):
        pltpu.make_async_copy(bufs[half], dst, ssems[half]).wait()

    def compute(half):
        buf = bufs[half]

        def p_body(p, carry):
            pv = [pos_v[p, pl.ds(c * L, L)] for c in range(EMBED_DIM // L)]
            for s in range(CB):
                for c in range(EMBED_DIM // L):
                    sl = pl.ds(c * L, L)
                    buf[s, p, sl] = buf[s, p, sl] * _SCALE + pv[c]
            return carry

        lax.fori_loop(0, SEQ_LEN, p_body, 0)

    def pair_body(p, scattered):
        b0 = pl.multiple_of(base + p * PAIR, 8)
        pltpu.sync_copy(idx_hbm.at[pl.ds(b0, PAIR)], idx_v)
        for half in range(2):
            dst = out_hbm.at[pl.ds(b0 + half * CB, CB)]

            @pl.when(scattered != 0)
            def _():
                drain_scatter(half, dst)

            gather_chunk(half, p)

        for half in range(2):
            dst = out_hbm.at[pl.ds(b0 + half * CB, CB)]
            drain_gather(half)
            compute(half)
            pltpu.async_copy(bufs[half], dst, ssems[half])
        return 1

    scattered = lax.fori_loop(0, N_PAIRS, pair_body, 0)

    @pl.when(scattered != 0)
    def _():
        last = pl.multiple_of(base + (N_PAIRS - 1) * PAIR, 8)
        for half in range(2):
            drain_scatter(half, out_hbm.at[pl.ds(last + half * CB, CB)])


@jax.jit
def _run(inputs, token_table, position_table):
    mesh = plsc.VectorSubcoreMesh(
        core_axis_name="c", subcore_axis_name="s", num_cores=NC, num_subcores=NS
    )
    kern = functools.partial(
        pl.kernel,
        out_type=jax.ShapeDtypeStruct((BATCH, SEQ_LEN, EMBED_DIM), jnp.float32),
        mesh=mesh,
        scratch_types=[
            pltpu.VMEM((SEQ_LEN, EMBED_DIM), jnp.float32),
            pltpu.VMEM((PAIR, SEQ_LEN), jnp.int32),
            pltpu.VMEM((CB, SEQ_LEN, EMBED_DIM), jnp.float32),
            pltpu.VMEM((CB, SEQ_LEN, EMBED_DIM), jnp.float32),
            pltpu.SemaphoreType.DMA,
            pltpu.SemaphoreType.DMA,
            pltpu.SemaphoreType.DMA,
            pltpu.SemaphoreType.DMA,
        ],
        compiler_params=pltpu.CompilerParams(use_tc_tiling_on_sc=False),
    )(_emb_kernel)
    return kern(inputs, token_table, position_table)


def kernel(inputs, token_table, position_table):
    return _run(inputs, token_table, position_table)
